# Initial kernel scaffold; baseline (speedup 1.0000x reference)
import jax
import jax.numpy as jnp
from jax.experimental import pallas as pl


def kernel(x, edge_index, W):
    raise NotImplementedError("write your pallas kernel here")


# speedup vs baseline: 215.9235x; 215.9235x over previous
import functools

import jax
import jax.numpy as jnp
from jax import lax
from jax.experimental import pallas as pl
from jax.experimental.pallas import tpu as pltpu
from jax.experimental.pallas import tpu_sc as plsc

f32 = jnp.float32
i32 = jnp.int32

NC = 2
NS = 16
NT = NC * NS
LANES = 16
EB = 2000

_MESH = plsc.VectorSubcoreMesh(core_axis_name="c", subcore_axis_name="s")


def _edge_pass(n_pad, e, first):
    C = n_pad // NS
    EPT = e // NT
    out_type = jax.ShapeDtypeStruct((2 * n_pad,), f32) if first else (
        jax.ShapeDtypeStruct((n_pad,), f32),
        jax.ShapeDtypeStruct((2 * n_pad,), f32),
    )

    scratch = [
        pltpu.VMEM_SHARED((n_pad,), f32),
        pltpu.VMEM_SHARED((n_pad,), f32),
        pltpu.VMEM((C,), f32),
        pltpu.VMEM((C,), f32),
        pltpu.VMEM((C,), f32),
        pltpu.VMEM((EB,), i32),
        pltpu.VMEM((EB,), i32),
        pltpu.VMEM((EB,), f32),
        pltpu.VMEM((LANES,), f32),
    ]

    def body(*refs):
        if first:
            (h_ref, src_ref, dst_ref, pout_ref,
             h_sh, agg_sh, hbuf, q0, q1, sbuf, dbuf, vbuf, wbuf) = refs
        else:
            (h_ref, p_ref, w_ref, src_ref, dst_ref, hout_ref, pout_ref,
             h_sh, agg_sh, hbuf, q0, q1, sbuf, dbuf, vbuf, wbuf) = refs
        c = lax.axis_index("c")
        s = lax.axis_index("s")
        base = s * C

        pltpu.sync_copy(h_ref.at[pl.ds(base, C)], hbuf)
        if not first:
            pltpu.sync_copy(w_ref, wbuf)
            pltpu.sync_copy(p_ref.at[pl.ds(base, C)], q0)
            pltpu.sync_copy(p_ref.at[pl.ds(n_pad + base, C)], q1)
            wv = wbuf[...]

            @pl.loop(0, C, step=LANES)
            def _(i):
                sl = pl.ds(i, LANES)
                hbuf[sl] = (hbuf[sl] + q0[sl] + q1[sl]) * wv
                q0[sl] = jnp.zeros((LANES,), f32)
        else:
            @pl.loop(0, C, step=LANES)
            def _(i):
                q0[pl.ds(i, LANES)] = jnp.zeros((LANES,), f32)

        pltpu.sync_copy(hbuf, h_sh.at[pl.ds(base, C)])
        pltpu.sync_copy(q0, agg_sh.at[pl.ds(base, C)])
        if not first:
            @pl.when(c == 0)
            def _():
                pltpu.sync_copy(hbuf, hout_ref.at[pl.ds(base, C)])

        plsc.subcore_barrier()

        ebase = (c * NS + s) * EPT

        @pl.loop(0, EPT, step=EB)
        def _(i):
            pltpu.sync_copy(src_ref.at[pl.ds(ebase + i, EB)], sbuf)
            pltpu.sync_copy(dst_ref.at[pl.ds(ebase + i, EB)], dbuf)
            pltpu.sync_copy(h_sh.at[sbuf], vbuf)
            pltpu.sync_copy(vbuf, agg_sh.at[dbuf], add=True)

        plsc.subcore_barrier()

        pltpu.sync_copy(agg_sh.at[pl.ds(base, C)], q1)
        pltpu.sync_copy(q1, pout_ref.at[pl.ds(c * n_pad + base, C)])

    return pl.kernel(body, out_type=out_type, mesh=_MESH,
                     scratch_types=scratch)


def _final_combine(n_pad):
    C = n_pad // NS

    scratch = [
        pltpu.VMEM((C,), f32),
        pltpu.VMEM((C,), f32),
        pltpu.VMEM((C,), f32),
        pltpu.VMEM((LANES,), f32),
    ]

    def body(h_ref, p_ref, w_ref, hout_ref, hbuf, q0, q1, wbuf):
        c = lax.axis_index("c")
        s = lax.axis_index("s")
        base = s * C

        @pl.when(c == 0)
        def _():
            pltpu.sync_copy(w_ref, wbuf)
            pltpu.sync_copy(h_ref.at[pl.ds(base, C)], hbuf)
            pltpu.sync_copy(p_ref.at[pl.ds(base, C)], q0)
            pltpu.sync_copy(p_ref.at[pl.ds(n_pad + base, C)], q1)
            wv = wbuf[...]

            @pl.loop(0, C, step=LANES)
            def _(i):
                sl = pl.ds(i, LANES)
                hbuf[sl] = (hbuf[sl] + q0[sl] + q1[sl]) * wv

            pltpu.sync_copy(hbuf, hout_ref.at[pl.ds(base, C)])

    return pl.kernel(body, out_type=jax.ShapeDtypeStruct((n_pad,), f32),
                     mesh=_MESH, scratch_types=scratch)


def kernel(x, edge_index, W):
    n = x.shape[0]
    e = edge_index.shape[1]
    num_hop = W.shape[0]
    n_pad = -(-n // (NS * LANES)) * (NS * LANES)
    assert e % (NT * EB) == 0

    src = edge_index[0].astype(i32)
    dst = edge_index[1].astype(i32)
    wv = [jnp.broadcast_to(W[k, 0, 0].astype(f32), (LANES,))
          for k in range(num_hop)]

    h = jnp.zeros((n_pad,), f32).at[:n].set(x[:, 0])

    p = _edge_pass(n_pad, e, True)(h, src, dst)
    for k in range(1, num_hop):
        h, p = _edge_pass(n_pad, e, False)(h, p, wv[k - 1], src, dst)
    h = _final_combine(n_pad)(h, p, wv[num_hop - 1])

    return h[:n].reshape(n, 1)


# speedup vs baseline: 398.7233x; 1.8466x over previous
import functools

import jax
import jax.numpy as jnp
from jax import lax
from jax.experimental import pallas as pl
from jax.experimental.pallas import tpu as pltpu
from jax.experimental.pallas import tpu_sc as plsc

f32 = jnp.float32
i32 = jnp.int32

NC = 2
NS = 16
NT = NC * NS
LANES = 16
EB = 2000

_MESH = plsc.VectorSubcoreMesh(core_axis_name="c", subcore_axis_name="s")


def _edge_pass(n_pad, e, first):
    C = n_pad // NS
    EPT = e // NT
    out_type = jax.ShapeDtypeStruct((2 * n_pad,), f32) if first else (
        jax.ShapeDtypeStruct((n_pad,), f32),
        jax.ShapeDtypeStruct((2 * n_pad,), f32),
    )

    NB = 4

    scratch = [
        pltpu.VMEM_SHARED((n_pad,), f32),
        pltpu.VMEM_SHARED((n_pad,), f32),
        pltpu.VMEM((C,), f32),
        pltpu.VMEM((C,), f32),
        pltpu.VMEM((C,), f32),
        [pltpu.VMEM((EB,), i32)] * NB,
        [pltpu.VMEM((EB,), i32)] * NB,
        [pltpu.VMEM((EB,), f32)] * NB,
        [pltpu.SemaphoreType.DMA] * NB,
        [pltpu.SemaphoreType.DMA] * NB,
        pltpu.VMEM((LANES,), f32),
    ]

    def body(*refs):
        if first:
            (h_ref, src_ref, dst_ref, pout_ref,
             h_sh, agg_sh, hbuf, q0, q1,
             sbufs, dbufs, vbufs, sl, ss, wbuf) = refs
        else:
            (h_ref, p_ref, w_ref, src_ref, dst_ref, hout_ref, pout_ref,
             h_sh, agg_sh, hbuf, q0, q1,
             sbufs, dbufs, vbufs, sl, ss, wbuf) = refs
        c = lax.axis_index("c")
        s = lax.axis_index("s")
        base = s * C

        pltpu.sync_copy(h_ref.at[pl.ds(base, C)], hbuf)
        if not first:
            pltpu.sync_copy(w_ref, wbuf)
            pltpu.sync_copy(p_ref.at[pl.ds(base, C)], q0)
            pltpu.sync_copy(p_ref.at[pl.ds(n_pad + base, C)], q1)
            wv = wbuf[...]

            @pl.loop(0, C, step=LANES)
            def _(i):
                sl = pl.ds(i, LANES)
                hbuf[sl] = (hbuf[sl] + q0[sl] + q1[sl]) * wv
                q0[sl] = jnp.zeros((LANES,), f32)
        else:
            @pl.loop(0, C, step=LANES)
            def _(i):
                q0[pl.ds(i, LANES)] = jnp.zeros((LANES,), f32)

        pltpu.sync_copy(hbuf, h_sh.at[pl.ds(base, C)])
        pltpu.sync_copy(q0, agg_sh.at[pl.ds(base, C)])
        if not first:
            @pl.when(c == 0)
            def _():
                pltpu.sync_copy(hbuf, hout_ref.at[pl.ds(base, C)])

        plsc.subcore_barrier()

        ebase = (c * NS + s) * EPT
        NBLK = EPT // EB
        assert NBLK >= 4 and (NBLK - 4) % 4 == 0

        def start_loads(blk, b):
            off = ebase + blk * EB
            pltpu.async_copy(src_ref.at[pl.ds(off, EB)], sbufs[b], sl[b])
            pltpu.async_copy(dst_ref.at[pl.ds(off, EB)], dbufs[b], sl[b])

        def wait_loads(b):
            pltpu.make_async_copy(
                src_ref.at[pl.ds(0, EB)], sbufs[b], sl[b]).wait()
            pltpu.make_async_copy(
                dst_ref.at[pl.ds(0, EB)], dbufs[b], sl[b]).wait()

        def gather(b):
            pltpu.sync_copy(h_sh.at[sbufs[b]], vbufs[b])

        def start_scatter(b):
            pltpu.async_copy(vbufs[b], agg_sh.at[dbufs[b]], ss[b], add=True)

        def wait_scatter(b):
            pltpu.make_async_copy(vbufs[b], agg_sh.at[dbufs[b]], ss[b]).wait()

        start_loads(0, 0)
        start_loads(1, 1)
        for blk in (0, 1):
            wait_loads(blk)
            gather(blk)
            start_scatter(blk)
            start_loads(blk + 2, blk + 2)

        @pl.loop(2, NBLK - 2, step=4)
        def _(g):
            for j in range(4):
                b = (2 + j) % 4
                wait_loads(b)
                gather(b)
                start_scatter(b)
                wait_scatter((b + 2) % 4)
                start_loads(g + j + 2, (b + 2) % 4)

        for blk in (NBLK - 2, NBLK - 1):
            b = blk % 4
            wait_loads(b)
            gather(b)
            start_scatter(b)
            wait_scatter((b + 2) % 4)
        wait_scatter((NBLK - 2) % 4)
        wait_scatter((NBLK - 1) % 4)

        plsc.subcore_barrier()

        pltpu.sync_copy(agg_sh.at[pl.ds(base, C)], q1)
        pltpu.sync_copy(q1, pout_ref.at[pl.ds(c * n_pad + base, C)])

    return pl.kernel(body, out_type=out_type, mesh=_MESH,
                     scratch_types=scratch)


def _final_combine(n_pad):
    C = n_pad // NS

    scratch = [
        pltpu.VMEM((C,), f32),
        pltpu.VMEM((C,), f32),
        pltpu.VMEM((C,), f32),
        pltpu.VMEM((LANES,), f32),
    ]

    def body(h_ref, p_ref, w_ref, hout_ref, hbuf, q0, q1, wbuf):
        c = lax.axis_index("c")
        s = lax.axis_index("s")
        base = s * C

        @pl.when(c == 0)
        def _():
            pltpu.sync_copy(w_ref, wbuf)
            pltpu.sync_copy(h_ref.at[pl.ds(base, C)], hbuf)
            pltpu.sync_copy(p_ref.at[pl.ds(base, C)], q0)
            pltpu.sync_copy(p_ref.at[pl.ds(n_pad + base, C)], q1)
            wv = wbuf[...]

            @pl.loop(0, C, step=LANES)
            def _(i):
                sl = pl.ds(i, LANES)
                hbuf[sl] = (hbuf[sl] + q0[sl] + q1[sl]) * wv

            pltpu.sync_copy(hbuf, hout_ref.at[pl.ds(base, C)])

    return pl.kernel(body, out_type=jax.ShapeDtypeStruct((n_pad,), f32),
                     mesh=_MESH, scratch_types=scratch)


def kernel(x, edge_index, W):
    n = x.shape[0]
    e = edge_index.shape[1]
    num_hop = W.shape[0]
    n_pad = -(-n // (NS * LANES)) * (NS * LANES)
    assert e % (NT * EB) == 0

    src = edge_index[0].astype(i32)
    dst = edge_index[1].astype(i32)
    wv = [jnp.broadcast_to(W[k, 0, 0].astype(f32), (LANES,))
          for k in range(num_hop)]

    h = jnp.zeros((n_pad,), f32).at[:n].set(x[:, 0])

    p = _edge_pass(n_pad, e, True)(h, src, dst)
    for k in range(1, num_hop):
        h, p = _edge_pass(n_pad, e, False)(h, p, wv[k - 1], src, dst)
    h = _final_combine(n_pad)(h, p, wv[num_hop - 1])

    return h[:n].reshape(n, 1)
